# Initial kernel scaffold; baseline (speedup 1.0000x reference)
import jax
import jax.numpy as jnp
from jax.experimental import pallas as pl


def kernel(inputs, embedding):
    raise NotImplementedError("write your pallas kernel here")


# speedup vs baseline: 2.6142x; 2.6142x over previous
import functools

import jax
import jax.numpy as jnp
from jax import lax
from jax.experimental import pallas as pl
from jax.experimental.pallas import tpu as pltpu
from jax.experimental.pallas import tpu_sc as plsc

_VOCAB = 1000000
_EMBED = 64
_BATCH = 16384
_HIST = 50

_CHUNK = 16
_LANES = 16


def _build_kernel(num_cores, num_subcores):
  nw = num_cores * num_subcores
  rows_per_w = _BATCH // nw
  nchunks = rows_per_w // _CHUNK
  gat = _CHUNK * _HIST

  mesh = plsc.VectorSubcoreMesh(
      core_axis_name="c", subcore_axis_name="s",
      num_cores=num_cores, num_subcores=num_subcores)

  @functools.partial(
      pl.kernel,
      out_type=jax.ShapeDtypeStruct((_BATCH, _EMBED), jnp.float32),
      mesh=mesh,
      scratch_types=[
          pltpu.VMEM((gat,), jnp.int32),
          pltpu.VMEM((gat,), jnp.int32),
          pltpu.VMEM((gat, _EMBED), jnp.float32),
          pltpu.VMEM((gat, _EMBED), jnp.float32),
          pltpu.VMEM((_CHUNK, _EMBED), jnp.float32),
          pltpu.SemaphoreType.DMA,
          pltpu.SemaphoreType.DMA,
      ],
      compiler_params=pltpu.CompilerParams(use_tc_tiling_on_sc=False),
  )
  def skipgram(idx_hbm, table_hbm, out_hbm,
               idx0, idx1, rows0, rows1, outb, sem0, sem1):
    wid = lax.axis_index("s") * num_cores + lax.axis_index("c")
    row0 = wid * rows_per_w
    idx_bufs = (idx0, idx1)
    row_bufs = (rows0, rows1)
    sems = (sem0, sem1)

    def fetch(g, slot):
      off = (row0 + g * _CHUNK) * _HIST
      pltpu.sync_copy(idx_hbm.at[pl.ds(off, gat)], idx_bufs[slot])
      pltpu.make_async_copy(
          table_hbm.at[idx_bufs[slot]], row_bufs[slot], sems[slot]).start()

    lane = jnp.arange(_LANES, dtype=jnp.int32)
    perms = [lane ^ sh for sh in (1, 2, 4, 8)]

    def xlane(v, op):
      for p in perms:
        v = op(v, v.at[p].get(mode="promise_in_bounds"))
      return v

    def compute(g, slot):
      rows = row_bufs[slot]
      pltpu.make_async_copy(
          table_hbm.at[idx_bufs[slot]], rows, sems[slot]).wait()

      @pl.loop(0, _CHUNK)
      def per_row(r):
        base = r * _HIST
        a0 = rows[base, pl.ds(0, _LANES)]
        a1 = rows[base, pl.ds(_LANES, _LANES)]
        a2 = rows[base, pl.ds(2 * _LANES, _LANES)]
        a3 = rows[base, pl.ds(3 * _LANES, _LANES)]
        for j in range(1, _HIST):
          a0 = a0 + rows[base + j, pl.ds(0, _LANES)]
          a1 = a1 + rows[base + j, pl.ds(_LANES, _LANES)]
          a2 = a2 + rows[base + j, pl.ds(2 * _LANES, _LANES)]
          a3 = a3 + rows[base + j, pl.ds(3 * _LANES, _LANES)]
        m = xlane(jnp.maximum(jnp.maximum(a0, a1), jnp.maximum(a2, a3)),
                  jnp.maximum)
        e0 = jnp.exp(a0 - m)
        e1 = jnp.exp(a1 - m)
        e2 = jnp.exp(a2 - m)
        e3 = jnp.exp(a3 - m)
        inv = 1.0 / xlane((e0 + e1) + (e2 + e3), jnp.add)
        outb[r, pl.ds(0, _LANES)] = e0 * inv
        outb[r, pl.ds(_LANES, _LANES)] = e1 * inv
        outb[r, pl.ds(2 * _LANES, _LANES)] = e2 * inv
        outb[r, pl.ds(3 * _LANES, _LANES)] = e3 * inv

      pltpu.sync_copy(outb, out_hbm.at[pl.ds(row0 + g * _CHUNK, _CHUNK)])

    fetch(0, 0)

    @pl.loop(0, nchunks, step=2)
    def pipeline(g):
      @pl.when(g + 1 < nchunks)
      def _():
        fetch(g + 1, 1)
      compute(g, 0)

      @pl.when(g + 1 < nchunks)
      def _():
        @pl.when(g + 2 < nchunks)
        def _():
          fetch(g + 2, 0)
        compute(g + 1, 1)

  return skipgram


@jax.jit
def kernel(inputs, embedding):
  info = plsc.get_sparse_core_info()
  sk = _build_kernel(info.num_cores, info.num_subcores)
  return sk(inputs.reshape(-1), embedding)
